# Initial kernel scaffold; baseline (speedup 1.0000x reference)
import jax
import jax.numpy as jnp
from jax.experimental import pallas as pl


def kernel(x):
    raise NotImplementedError("write your pallas kernel here")


# speedup vs baseline: 80.9685x; 80.9685x over previous
import functools

import jax
import jax.numpy as jnp
from jax import lax
from jax.experimental import pallas as pl
from jax.experimental.pallas import tpu as pltpu

_Q = 1024
_R = 8
_L = 512
_D = 147
_TEMP = 5.0
_ITERS = 10


def _soft_kmeans_body(x_ref, rec_ref, labels_ref):
    x = x_ref[0]

    def dist(c):
        xc = lax.dot_general(
            x, c, (((1,), (1,)), ((), ())),
            preferred_element_type=jnp.float32,
            precision=lax.Precision.DEFAULT,
        )
        c2 = jnp.sum(c * c, axis=1)[None, :]
        x2 = jnp.sum(x * x, axis=1)[:, None]
        return x2 - 2.0 * xc + c2

    def step(_, c):
        d = dist(c)
        z = -_TEMP * d
        z = z - jnp.max(z, axis=1, keepdims=True)
        e = jnp.exp(z)
        p = e / jnp.sum(e, axis=1, keepdims=True)
        w = p / (jnp.sum(p, axis=0, keepdims=True) + 1e-9)
        new_c = lax.dot_general(
            w, x, (((0,), (0,)), ((), ())),
            preferred_element_type=jnp.float32,
            precision=lax.Precision.DEFAULT,
        )
        return new_c

    c = lax.fori_loop(0, _ITERS, step, x[:_L, :])

    d = dist(c)
    m = jnp.min(d, axis=1, keepdims=True)
    li = lax.broadcasted_iota(jnp.int32, (_Q, _L), 1)
    lab = jnp.min(jnp.where(d == m, li, _L), axis=1, keepdims=True)
    labels_ref[0] = lab + pl.program_id(0) * _L

    onehot = (li == lab).astype(jnp.float32)
    rec_ref[0] = lax.dot_general(
        onehot, c, (((1,), (0,)), ((), ())),
        preferred_element_type=jnp.float32,
        precision=lax.Precision.HIGHEST,
    )


def _run_soft_kmeans(xr):
    return pl.pallas_call(
        _soft_kmeans_body,
        grid=(_R,),
        in_specs=[pl.BlockSpec((1, _Q, _D), lambda r: (r, 0, 0))],
        out_specs=[
            pl.BlockSpec((1, _Q, _D), lambda r: (r, 0, 0)),
            pl.BlockSpec((1, _Q, 1), lambda r: (r, 0, 0)),
        ],
        out_shape=[
            jax.ShapeDtypeStruct((_R, _Q, _D), jnp.float32),
            jax.ShapeDtypeStruct((_R, _Q, 1), jnp.int32),
        ],
        compiler_params=pltpu.CompilerParams(
            dimension_semantics=("parallel",),
        ),
    )(xr)


def kernel(x):
    B, T, F = x.shape
    xr = x.reshape(_R, _Q, _D)
    rec, _ = _run_soft_kmeans(xr)
    return rec.reshape(B, T, F)
